# baseline (speedup 1.0000x reference)
import numpy as np
import jax
import jax.numpy as jnp
from jax import lax
from jax.experimental import pallas as pl
from jax.experimental.pallas import tpu as pltpu

_QUANTILES = (0.1, 0.25, 0.5, 0.75, 0.9)
_N_ROWS = 8192
_MINT = np.int32(-(2 ** 31))
_MAXT = np.int32(2 ** 31 - 1)
_FLIP = np.int32(0x7FFFFFFF)
_MAX16 = np.int16(32767)

_POS = [np.float32(q) * np.float32(_N_ROWS - 1) for q in _QUANTILES]
_KS = [int(np.floor(p)) for p in _POS]
_FRACS = [np.float32(p - np.floor(p)) for p in _POS]


def _count_mm(mask, ones_row):
    m = mask.astype(jnp.bfloat16)
    return lax.dot_general(
        ones_row, m, (((1,), (0,)), ((), ())),
        preferred_element_type=jnp.float32)


def _select_kernel(x_ref, o_ref):
    x = x_ref[...]
    n, c = x.shape
    ones_row = jnp.ones((1, n), jnp.bfloat16)
    i32 = lax.bitcast_convert_type(x, jnp.int32)
    s = jnp.where(i32 >= 0, i32, i32 ^ _FLIP)
    s_hi = (s >> 16).astype(jnp.int16)
    s_lo = ((s & jnp.int32(0xFFFF)) - jnp.int32(32768)).astype(jnp.int16)

    def hi_step(it, ps):
        b = 31 - it
        bit = lax.shift_left(jnp.int32(1), b)
        new_ps = []
        for r in range(5):
            t = ps[r] | bit
            t16 = ((t ^ _MINT) >> 16).astype(jnp.int16)
            cnt = _count_mm(s_hi < t16, ones_row)
            new_ps.append(jnp.where(cnt <= np.float32(_KS[r]), t, ps[r]))
        return tuple(new_ps)

    ps0 = tuple(jnp.zeros((1, c), jnp.int32) for _ in range(5))
    ps = lax.fori_loop(0, 16, hi_step, ps0)

    zs = []
    kps = []
    for r in range(5):
        p16 = ((ps[r] ^ _MINT) >> 16).astype(jnp.int16)
        cnt_hi = _count_mm(s_hi < p16, ones_row)
        kps.append(np.float32(_KS[r]) - cnt_hi)
        zs.append(jnp.where(s_hi == p16, s_lo, _MAX16))

    def lo_step(it, pls):
        b = 15 - it
        bit = lax.shift_left(jnp.int32(1), b)
        new_pls = []
        for r in range(5):
            t = pls[r] | bit
            t16 = (t - jnp.int32(32768)).astype(jnp.int16)
            cnt = _count_mm(zs[r] < t16, ones_row)
            new_pls.append(jnp.where(cnt <= kps[r], t, pls[r]))
        return tuple(new_pls)

    pls = lax.fori_loop(0, 16, lo_step, tuple(
        jnp.zeros((1, c), jnp.int32) for _ in range(5)))

    rows = []
    for r in range(5):
        sk = (ps[r] | pls[r]) ^ _MINT
        gt = s > sk
        le = np.float32(_N_ROWS) - _count_mm(gt, ones_row)
        nxt = jnp.min(jnp.where(gt, s, _MAXT), axis=0, keepdims=True)
        sk1 = jnp.where(le >= np.float32(_KS[r] + 2), sk, nxt)
        f_lo = lax.bitcast_convert_type(
            jnp.where(sk >= 0, sk, sk ^ _FLIP), jnp.float32)
        f_hi = lax.bitcast_convert_type(
            jnp.where(sk1 >= 0, sk1, sk1 ^ _FLIP), jnp.float32)
        rows.append(f_lo + _FRACS[r] * (f_hi - f_lo))
    o_ref[...] = jnp.concatenate(rows, axis=0)


def kernel(inputs):
    x = inputs
    n, d = x.shape
    cb = 256
    out = pl.pallas_call(
        _select_kernel,
        grid=(d // cb,),
        in_specs=[pl.BlockSpec((n, cb), lambda j: (0, j))],
        out_specs=pl.BlockSpec((5, cb), lambda j: (0, j)),
        out_shape=jax.ShapeDtypeStruct((5, d), jnp.float32),
        compiler_params=pltpu.CompilerParams(
            dimension_semantics=("parallel",),
        ),
    )(x)
    return out.T


# speedup vs baseline: 1.9975x; 1.9975x over previous
import numpy as np
import jax
import jax.numpy as jnp
from jax import lax
from jax.experimental import pallas as pl
from jax.experimental.pallas import tpu as pltpu

_QUANTILES = (0.1, 0.25, 0.5, 0.75, 0.9)
_N_ROWS = 8192
_MINT = np.int32(-(2 ** 31))
_MAXT = np.int32(2 ** 31 - 1)
_FLIP = np.int32(0x7FFFFFFF)
_MAX16 = np.int16(32767)

_POS = [np.float32(q) * np.float32(_N_ROWS - 1) for q in _QUANTILES]
_KS = [int(np.floor(p)) for p in _POS]
_FRACS = [np.float32(p - np.floor(p)) for p in _POS]


def _count16(mask):
    m = mask
    while m.shape[0] > 8:
        h = m.shape[0] // 2
        m = m[:h] + m[h:]
    return jnp.sum(m, axis=0, keepdims=True, dtype=jnp.int32)


def _select_kernel(x_ref, o_ref):
    x = x_ref[...]
    n, c = x.shape
    i32 = lax.bitcast_convert_type(x, jnp.int32)
    s = jnp.where(i32 >= 0, i32, i32 ^ _FLIP)
    s_hi = (s >> 16).astype(jnp.int16)
    s_lo = ((s & jnp.int32(0xFFFF)) - jnp.int32(32768)).astype(jnp.int16)

    def hi_step(it, carry):
        ps, bcs = carry
        b = 31 - it
        bit = lax.shift_left(jnp.int32(1), b)
        new_ps, new_bcs = [], []
        for r in range(5):
            t = ps[r] | bit
            t16 = ((t ^ _MINT) >> 16).astype(jnp.int16)
            cnt = _count16((s_hi < t16).astype(jnp.int16))
            acc = cnt <= _KS[r]
            new_ps.append(jnp.where(acc, t, ps[r]))
            new_bcs.append(jnp.where(acc, cnt, bcs[r]))
        return (tuple(new_ps), tuple(new_bcs))

    zeros = tuple(jnp.zeros((1, c), jnp.int32) for _ in range(5))
    ps, bcs = lax.fori_loop(0, 16, hi_step, (zeros, zeros))

    zs = []
    kps = []
    for r in range(5):
        p16 = ((ps[r] ^ _MINT) >> 16).astype(jnp.int16)
        kps.append(_KS[r] - bcs[r])
        zs.append(jnp.where(s_hi == p16, s_lo, _MAX16))

    def lo_step(it, pls):
        b = 15 - it
        bit = lax.shift_left(jnp.int32(1), b)
        new_pls = []
        for r in range(5):
            t = pls[r] | bit
            t16 = (t - jnp.int32(32768)).astype(jnp.int16)
            cnt = _count16((zs[r] < t16).astype(jnp.int16))
            new_pls.append(jnp.where(cnt <= kps[r], t, pls[r]))
        return tuple(new_pls)

    pls = lax.fori_loop(0, 16, lo_step, zeros)

    rows = []
    for r in range(5):
        sk = (ps[r] | pls[r]) ^ _MINT
        gt = s > sk
        le = _N_ROWS - _count16(gt.astype(jnp.int16))
        nxt = jnp.min(jnp.where(gt, s, _MAXT), axis=0, keepdims=True)
        sk1 = jnp.where(le >= _KS[r] + 2, sk, nxt)
        f_lo = lax.bitcast_convert_type(
            jnp.where(sk >= 0, sk, sk ^ _FLIP), jnp.float32)
        f_hi = lax.bitcast_convert_type(
            jnp.where(sk1 >= 0, sk1, sk1 ^ _FLIP), jnp.float32)
        rows.append(f_lo + _FRACS[r] * (f_hi - f_lo))
    o_ref[...] = jnp.concatenate(rows, axis=0)


def kernel(inputs):
    x = inputs
    n, d = x.shape
    cb = 256
    out = pl.pallas_call(
        _select_kernel,
        grid=(d // cb,),
        in_specs=[pl.BlockSpec((n, cb), lambda j: (0, j))],
        out_specs=pl.BlockSpec((5, cb), lambda j: (0, j)),
        out_shape=jax.ShapeDtypeStruct((5, d), jnp.float32),
        compiler_params=pltpu.CompilerParams(
            dimension_semantics=("parallel",),
        ),
    )(x)
    return out.T


# speedup vs baseline: 2.0912x; 1.0469x over previous
import numpy as np
import jax
import jax.numpy as jnp
from jax import lax
from jax.experimental import pallas as pl
from jax.experimental.pallas import tpu as pltpu

_QUANTILES = (0.1, 0.25, 0.5, 0.75, 0.9)
_N_ROWS = 8192
_MINT = np.int32(-(2 ** 31))
_MAXT = np.int32(2 ** 31 - 1)
_FLIP = np.int32(0x7FFFFFFF)
_MAX16 = np.int16(32767)

_POS = [np.float32(q) * np.float32(_N_ROWS - 1) for q in _QUANTILES]
_KS = [int(np.floor(p)) for p in _POS]
_FRACS = [np.float32(p - np.floor(p)) for p in _POS]


def _count16(mask):
    m = mask
    while m.shape[0] > 8:
        h = m.shape[0] // 2
        m = m[:h] + m[h:]
    return jnp.sum(m, axis=0, keepdims=True, dtype=jnp.int32)


def _select_kernel(x_ref, o_ref):
    x = x_ref[...]
    n, c = x.shape
    i32 = lax.bitcast_convert_type(x, jnp.int32)
    s = jnp.where(i32 >= 0, i32, i32 ^ _FLIP)
    s_hi = (s >> 16).astype(jnp.int16)
    s_lo = ((s & jnp.int32(0xFFFF)) - jnp.int32(32768)).astype(jnp.int16)

    def hi_step(it, carry):
        ps, bcs = carry
        b = 31 - it
        bit = lax.shift_left(jnp.int32(1), b)
        new_ps, new_bcs = [], []
        for r in range(5):
            t = ps[r] | bit
            t16 = ((t ^ _MINT) >> 16).astype(jnp.int16)
            cnt = _count16((s_hi < t16).astype(jnp.int16))
            acc = cnt <= _KS[r]
            new_ps.append(jnp.where(acc, t, ps[r]))
            new_bcs.append(jnp.where(acc, cnt, bcs[r]))
        return (tuple(new_ps), tuple(new_bcs))

    zeros = tuple(jnp.zeros((1, c), jnp.int32) for _ in range(5))
    ps, bcs = lax.fori_loop(0, 16, hi_step, (zeros, zeros))

    zs = []
    kps = []
    for r in range(5):
        p16 = ((ps[r] ^ _MINT) >> 16).astype(jnp.int16)
        kps.append(_KS[r] - bcs[r])
        zs.append(jnp.where(s_hi == p16, s_lo, _MAX16))

    def lo_step(it, pls):
        b = 15 - it
        bit = lax.shift_left(jnp.int32(1), b)
        new_pls = []
        for r in range(5):
            t = pls[r] | bit
            t16 = (t - jnp.int32(32768)).astype(jnp.int16)
            cnt = _count16((zs[r] < t16).astype(jnp.int16))
            new_pls.append(jnp.where(cnt <= kps[r], t, pls[r]))
        return tuple(new_pls)

    pls = lax.fori_loop(0, 16, lo_step, zeros)

    rows = []
    for r in range(5):
        sk = (ps[r] | pls[r]) ^ _MINT
        gt = s > sk
        le = _N_ROWS - _count16(gt.astype(jnp.int16))
        nxt = jnp.min(jnp.where(gt, s, _MAXT), axis=0, keepdims=True)
        sk1 = jnp.where(le >= _KS[r] + 2, sk, nxt)
        f_lo = lax.bitcast_convert_type(
            jnp.where(sk >= 0, sk, sk ^ _FLIP), jnp.float32)
        f_hi = lax.bitcast_convert_type(
            jnp.where(sk1 >= 0, sk1, sk1 ^ _FLIP), jnp.float32)
        rows.append(f_lo + _FRACS[r] * (f_hi - f_lo))
    o_ref[...] = jnp.concatenate(rows, axis=0)


def kernel(inputs):
    x = inputs
    n, d = x.shape
    cb = 128
    out = pl.pallas_call(
        _select_kernel,
        grid=(d // cb,),
        in_specs=[pl.BlockSpec((n, cb), lambda j: (0, j))],
        out_specs=pl.BlockSpec((5, cb), lambda j: (0, j)),
        out_shape=jax.ShapeDtypeStruct((5, d), jnp.float32),
        compiler_params=pltpu.CompilerParams(
            dimension_semantics=("parallel",),
        ),
    )(x)
    return out.T
